# baseline (speedup 1.0000x reference)
import jax
import jax.numpy as jnp
from jax.experimental import pallas as pl

NS = 0.2


def _bias_add_kernel(x_ref, b_ref, o_ref):
    o_ref[...] = x_ref[...] + b_ref[...]


def _bias_add(x, b):
    n, d = x.shape
    blk = 1000
    return pl.pallas_call(
        _bias_add_kernel,
        grid=(n // blk,),
        in_specs=[
            pl.BlockSpec((blk, d), lambda i: (i, 0)),
            pl.BlockSpec((1, d), lambda i: (0, 0)),
        ],
        out_specs=pl.BlockSpec((blk, d), lambda i: (i, 0)),
        out_shape=jax.ShapeDtypeStruct((n, d), x.dtype),
    )(x, b.reshape(1, d))


def _add_self_loops(src, dst, ea, n):
    cnt = jax.ops.segment_sum(jnp.ones(src.shape[0], dtype=jnp.float32), dst, num_segments=n)
    mean_ea = jax.ops.segment_sum(ea, dst, num_segments=n) / jnp.maximum(cnt, 1.0)[:, None]
    loop = jnp.arange(n, dtype=src.dtype)
    src2 = jnp.concatenate([src, loop])
    dst2 = jnp.concatenate([dst, loop])
    ea2 = jnp.concatenate([ea, mean_ea], axis=0)
    return src2, dst2, ea2


def _gatv2(x, src, dst, ea, Wl, bl, Wr, br, We, att, bias, H, C, concat):
    n = x.shape[0]
    xl = (x @ Wl + bl).reshape(n, H, C)
    xr = (x @ Wr + br).reshape(n, H, C)
    e = (ea @ We).reshape(-1, H, C)
    z = jax.nn.leaky_relu(xl[src] + xr[dst] + e, NS)
    alpha = jnp.sum(z * att, axis=-1)
    amax = jax.ops.segment_max(alpha, dst, num_segments=n)
    amax = jnp.where(jnp.isfinite(amax), amax, 0.0)
    ex = jnp.exp(alpha - amax[dst])
    den = jax.ops.segment_sum(ex, dst, num_segments=n)
    a = ex / (den[dst] + 1e-16)
    out = jax.ops.segment_sum(xl[src] * a[:, :, None], dst, num_segments=n)
    if concat:
        out = out.reshape(n, H * C)
    else:
        out = jnp.mean(out, axis=1)
    return _bias_add(out, bias)


def kernel(x, edge_index, edge_attr, Wl1, bl1, We1, att1, bias1, Wl2, bl2, We2, att2, bias2, Wl3, bl3, Wr3, br3, We3, att3, bias3):
    src, dst, ea = _add_self_loops(edge_index[0], edge_index[1], edge_attr, x.shape[0])
    h = _gatv2(x, src, dst, ea, Wl1, bl1, Wl1, bl1, We1, att1, bias1, 8, 32, True)
    h = jax.nn.elu(h)
    h = _gatv2(h, src, dst, ea, Wl2, bl2, Wl2, bl2, We2, att2, bias2, 8, 32, True)
    h = jax.nn.elu(h)
    out = _gatv2(h, src, dst, ea, Wl3, bl3, Wr3, br3, We3, att3, bias3, 1, 256, False)
    return out


# speedup vs baseline: 3.2692x; 3.2692x over previous
import functools

import jax
import jax.numpy as jnp
from jax import lax
from jax.experimental import pallas as pl
from jax.experimental.pallas import tpu as pltpu
from jax.experimental.pallas import tpu_sc as plsc

N = 10000
NP = 10240
E = 320000
E2 = E + N
E2P = 331776
EXT = E2P + 64
DF = 128
HID = 256
HALF = 128
NS = 0.2
K = 64
f32 = jnp.float32
i32 = jnp.int32

_mesh = plsc.VectorSubcoreMesh(core_axis_name="c", subcore_axis_name="s")
_cp = pltpu.CompilerParams(needs_layout_passes=False)
NEG = -3.0e38


def _iota():
    return lax.iota(i32, 16)


def _sget(ref, idx):
    v = plsc.load_gather(ref, [jnp.full((16,), idx, i32)])
    return jnp.max(v)


@functools.partial(
    pl.kernel, mesh=_mesh, compiler_params=_cp,
    out_type=jax.ShapeDtypeStruct((NP, 16), f32),
    scratch_types=[
        pltpu.VMEM((40,), i32),
        pltpu.VMEM((K,), i32),
        pltpu.VMEM((K, 16), f32),
        pltpu.VMEM((320, 16), f32),
    ],
)
def _k0(ea_ref, dst_ref, offs_ref, mean_ref, offs_v, dst_v, buf_v, nb_v):
    core = lax.axis_index("c")
    sub = lax.axis_index("s")
    wid = core * 16 + sub
    iota = _iota()
    zero16 = jnp.zeros((16,), f32)

    def zrow(j, _):
        rv = jnp.full((16,), j, i32)
        plsc.store_scatter(nb_v, [rv, iota], zero16)
        return 0

    lax.fori_loop(0, 320, zrow, 0)

    pltpu.sync_copy(offs_ref, offs_v)
    estart = _sget(offs_v, wid)
    eend = _sget(offs_v, wid + 1)
    nbase = wid * 320
    e0al = (estart // K) * K
    nch = (eend - e0al + K - 1) // K

    def chunk(ci, _):
        e0 = e0al + ci * K
        pltpu.sync_copy(ea_ref.at[pl.ds(e0, K)], buf_v)
        pltpu.sync_copy(dst_ref.at[pl.ds(e0, K)], dst_v)
        lo = jnp.maximum(estart - e0, 0)
        hi = jnp.minimum(eend - e0, K)

        def edge(r, _):
            rv = jnp.full((16,), r, i32)
            erow = plsc.load_gather(buf_v, [rv, iota])
            d = jnp.max(plsc.load_gather(dst_v, [rv]))
            lv = jnp.full((16,), d - nbase, i32)
            cur = plsc.load_gather(nb_v, [lv, iota])
            plsc.store_scatter(nb_v, [lv, iota], cur + erow)
            return 0

        lax.fori_loop(lo, hi, edge, 0)
        return 0

    lax.fori_loop(0, nch, chunk, 0)

    def fin(j, _):
        rv = jnp.full((16,), j, i32)
        row = plsc.load_gather(nb_v, [rv, iota])
        cnt = plsc.load_gather(nb_v, [rv, jnp.full((16,), 4, i32)])
        r = 1.0 / jnp.maximum(cnt, 1.0)
        plsc.store_scatter(nb_v, [rv, iota], row * r)
        return 0

    lax.fori_loop(0, 320, fin, 0)
    for j in range(5):
        pltpu.sync_copy(nb_v.at[pl.ds(j * 64, 64)],
                        mean_ref.at[pl.ds(nbase + j * 64, 64)])


def _mm_first(x, W, b):
    def body(x_ref, w_ref, b_ref, o_ref):
        o_ref[...] = (jnp.dot(x_ref[...], w_ref[...],
                              preferred_element_type=f32) + b_ref[...])[None]

    return pl.pallas_call(
        body,
        grid=(2, 10),
        in_specs=[
            pl.BlockSpec((1000, DF), lambda j, i: (i, 0)),
            pl.BlockSpec((DF, HALF), lambda j, i: (0, j)),
            pl.BlockSpec((1, HALF), lambda j, i: (0, j)),
        ],
        out_specs=pl.BlockSpec((1, 1000, HALF), lambda j, i: (j, i, 0)),
        out_shape=jax.ShapeDtypeStruct((2, N, HALF), f32),
    )(x, W, b.reshape(1, HID)).reshape(2 * N, HALF)


def _mm_mid(hflat, Ws, bs):
    h3 = hflat.reshape(2, NP, HALF)
    nout = len(Ws)

    def body(*refs):
        a_ref, b_ref = refs[0], refs[1]
        xa = jnp.concatenate([a_ref[0], b_ref[0]], axis=1)
        xa = jnp.where(xa > 0, xa, jnp.exp(jnp.minimum(xa, 0.0)) - 1.0)
        for t in range(nout):
            w_ref = refs[2 + 2 * t]
            bb_ref = refs[3 + 2 * t]
            o_ref = refs[2 + 2 * nout + t]
            o_ref[...] = (jnp.dot(xa, w_ref[...],
                                  preferred_element_type=f32) + bb_ref[...])[None]

    in_specs = [
        pl.BlockSpec((1, 1000, HALF), lambda j, i: (0, i, 0)),
        pl.BlockSpec((1, 1000, HALF), lambda j, i: (1, i, 0)),
    ]
    args = [h3, h3]
    for (W, b) in zip(Ws, bs):
        in_specs.append(pl.BlockSpec((HID, HALF), lambda j, i: (0, j)))
        in_specs.append(pl.BlockSpec((1, HALF), lambda j, i: (0, j)))
        args.append(W)
        args.append(b.reshape(1, HID))
    outs = pl.pallas_call(
        body,
        grid=(2, 10),
        in_specs=in_specs,
        out_specs=[pl.BlockSpec((1, 1000, HALF), lambda j, i: (j, i, 0))] * nout,
        out_shape=[jax.ShapeDtypeStruct((2, N, HALF), f32)] * nout,
    )(*args)
    if nout == 1:
        return outs[0].reshape(2 * N, HALF)
    return [o.reshape(2 * N, HALF) for o in outs]


def _mm_edge(ea2p, Wep):
    def body(a_ref, w_ref, o_ref):
        o_ref[...] = jnp.dot(a_ref[...], w_ref[...],
                             preferred_element_type=f32)[None]

    return pl.pallas_call(
        body,
        grid=(2, E2P // 2048),
        in_specs=[
            pl.BlockSpec((2048, 16), lambda j, i: (i, 0)),
            pl.BlockSpec((16, HALF), lambda j, i: (0, j)),
        ],
        out_specs=pl.BlockSpec((1, 2048, HALF), lambda j, i: (j, i, 0)),
        out_shape=jax.ShapeDtypeStruct((2, E2P, HALF), f32),
    )(ea2p, Wep).reshape(2 * E2P, HALF)


def _make_k2(nh, cph):
    @functools.partial(
        pl.kernel, mesh=_mesh, compiler_params=_cp,
        out_type=[
            jax.ShapeDtypeStruct((2 * E2P, 4), f32),
            jax.ShapeDtypeStruct((128, 16), f32),
        ],
        scratch_types=[
            pltpu.VMEM((K,), i32),
            pltpu.VMEM((K,), i32),
            pltpu.VMEM((K,), i32),
            pltpu.VMEM((K, HALF), f32),
            pltpu.VMEM((K, HALF), f32),
            pltpu.VMEM((K, HALF), f32),
            pltpu.VMEM((HALF, 16), f32),
            pltpu.VMEM((K, 4), f32),
            pltpu.VMEM((4, 16), f32),
            pltpu.SemaphoreType.DMA,
            pltpu.SemaphoreType.DMA,
            pltpu.SemaphoreType.DMA,
        ],
    )
    def k2(xl_ref, xr_ref, es_ref, src_ref, dstg_ref, perm_ref, att_ref,
           alpha_ref, mx_ref,
           idxs_v, idxd_v, idxp_v, xls_v, xrs_v, es_v, attb_v, stga_v,
           stgm_v, sem1, sem2, sem3):
        core = lax.axis_index("c")
        sub = lax.axis_index("s")
        iota = _iota()
        coff = jnp.full((16,), core * N, i32)
        eoff = jnp.full((16,), core * E2P, i32)
        pltpu.sync_copy(att_ref.at[pl.ds(core * HALF, HALF)], attb_v)

        base_e = sub * (E2P // 16)
        nch = E2P // 16 // K

        def chunk(ci, m):
            e0 = base_e + ci * K
            pltpu.sync_copy(src_ref.at[pl.ds(e0, K)], idxs_v)
            pltpu.sync_copy(dstg_ref.at[pl.ds(e0, K)], idxd_v)
            pltpu.sync_copy(perm_ref.at[pl.ds(e0, K)], idxp_v)
            for g in range(K // 16):
                sl = pl.ds(g * 16, 16)
                idxs_v[sl] = idxs_v[sl] + coff
                idxd_v[sl] = idxd_v[sl] + coff
                idxp_v[sl] = idxp_v[sl] + eoff
            cp1 = pltpu.async_copy(xl_ref.at[idxs_v], xls_v, sem1)
            cp2 = pltpu.async_copy(xr_ref.at[idxd_v], xrs_v, sem2)
            cp3 = pltpu.async_copy(es_ref.at[idxp_v], es_v, sem3)
            cp1.wait()
            cp2.wait()
            cp3.wait()

            def group(g, mg):
                rows = iota + g * 16
                accs = [jnp.zeros((16,), f32) for _ in range(nh)]
                for c in range(HALF):
                    colv = jnp.full((16,), c, i32)
                    va = plsc.load_gather(xls_v, [rows, colv])
                    vb = plsc.load_gather(xrs_v, [rows, colv])
                    ve = plsc.load_gather(es_v, [rows, colv])
                    s = va + vb + ve
                    ly = jnp.maximum(s, s * NS)
                    accs[c // cph] = accs[c // cph] + ly * attb_v[c, :]
                mnew = []
                for lh in range(nh):
                    plsc.store_scatter(stga_v, [rows, jnp.full((16,), lh, i32)],
                                       accs[lh])
                    mnew.append(jnp.maximum(mg[lh], accs[lh]))
                return tuple(mnew)

            m = lax.fori_loop(0, K // 16, group, m)
            pltpu.sync_copy(stga_v, alpha_ref.at[pl.ds(core * E2P + e0, K)])
            return m

        m0 = tuple(jnp.full((16,), NEG, f32) for _ in range(nh))
        m = lax.fori_loop(0, nch, chunk, m0)
        for lh in range(nh):
            stgm_v[lh, :] = m[lh]
        for lh in range(nh, 4):
            stgm_v[lh, :] = jnp.full((16,), NEG, f32)
        wid = core * 16 + sub
        pltpu.sync_copy(stgm_v, mx_ref.at[pl.ds(wid * 4, 4)])

    return k2


def _make_k3(split):
    @functools.partial(
        pl.kernel, mesh=_mesh, compiler_params=_cp,
        out_type=[
            jax.ShapeDtypeStruct((EXT, 16), f32),
            jax.ShapeDtypeStruct((NP, 16), f32),
        ],
        scratch_types=[
            pltpu.VMEM((40,), i32),
            pltpu.VMEM((K,), i32),
            pltpu.VMEM((K, 4), f32),
            pltpu.VMEM((K, 4), f32),
            pltpu.VMEM((K, 16), f32),
            pltpu.VMEM((128, 16), f32),
            pltpu.VMEM((320, 16), f32),
        ],
    )
    def k3(alpha_ref, dsts_ref, mx_ref, offs_ref,
           ex_ref, den_ref,
           offs_v, dst_v, a0_v, a1_v, stg_v, mx_v, den_v):
        core = lax.axis_index("c")
        sub = lax.axis_index("s")
        iota = _iota()
        wid = core * 16 + sub
        zero16 = jnp.zeros((16,), f32)

        pltpu.sync_copy(mx_ref, mx_v)
        if split:
            Mv = []
            for h in range(8):
                acc = jnp.full((16,), NEG, f32)
                for w in range(16):
                    acc = jnp.maximum(acc, mx_v[((h // 4) * 16 + w) * 4 + (h % 4), :])
                Mv.append(jnp.full((16,), jnp.max(acc), f32))
        else:
            acc0 = jnp.full((16,), NEG, f32)
            acc1 = jnp.full((16,), NEG, f32)
            for w in range(16):
                acc0 = jnp.maximum(acc0, mx_v[w * 4, :])
                acc1 = jnp.maximum(acc1, mx_v[(16 + w) * 4, :])
            Mv = [jnp.full((16,), jnp.max(acc0) + jnp.max(acc1), f32)]

        def zrow(j, _):
            rv = jnp.full((16,), j, i32)
            plsc.store_scatter(den_v, [rv, iota], zero16)
            return 0

        lax.fori_loop(0, 320, zrow, 0)
        for i in range(K):
            stg_v[i, :] = zero16

        pltpu.sync_copy(offs_ref, offs_v)
        estart = _sget(offs_v, wid)
        eend = _sget(offs_v, wid + 1)
        nbase = wid * 320
        e0al = (estart // K) * K
        nch = (eend - e0al + K - 1) // K

        def chunk(ci, _):
            e0 = e0al + ci * K
            pltpu.sync_copy(dsts_ref.at[pl.ds(e0, K)], dst_v)
            pltpu.sync_copy(alpha_ref.at[pl.ds(e0, K)], a0_v)
            pltpu.sync_copy(alpha_ref.at[pl.ds(E2P + e0, K)], a1_v)

            def group(g, _):
                rows = iota + g * 16
                if split:
                    for lh in range(4):
                        colv = jnp.full((16,), lh, i32)
                        v0 = plsc.load_gather(a0_v, [rows, colv])
                        v1 = plsc.load_gather(a1_v, [rows, colv])
                        ex0 = jnp.exp(v0 - Mv[lh])
                        ex1 = jnp.exp(v1 - Mv[4 + lh])
                        plsc.store_scatter(stg_v, [rows, colv], ex0)
                        plsc.store_scatter(
                            stg_v, [rows, jnp.full((16,), 4 + lh, i32)], ex1)
                else:
                    colv = jnp.full((16,), 0, i32)
                    v0 = plsc.load_gather(a0_v, [rows, colv])
                    v1 = plsc.load_gather(a1_v, [rows, colv])
                    plsc.store_scatter(stg_v, [rows, colv],
                                       jnp.exp(v0 + v1 - Mv[0]))
                return 0

            lax.fori_loop(0, K // 16, group, 0)
            pltpu.sync_copy(stg_v, ex_ref.at[pl.ds(e0, K)])

            lo = jnp.maximum(estart - e0, 0)
            hi = jnp.minimum(eend - e0, K)

            def edge(r, _):
                rv = jnp.full((16,), r, i32)
                evec = plsc.load_gather(stg_v, [rv, iota])
                d = jnp.max(plsc.load_gather(dst_v, [rv]))
                lv = jnp.full((16,), d - nbase, i32)
                cur = plsc.load_gather(den_v, [lv, iota])
                plsc.store_scatter(den_v, [lv, iota], cur + evec)
                return 0

            lax.fori_loop(lo, hi, edge, 0)
            return 0

        lax.fori_loop(0, nch, chunk, 0)
        for j in range(5):
            pltpu.sync_copy(den_v.at[pl.ds(j * 64, 64)],
                            den_ref.at[pl.ds(nbase + j * 64, 64)])

    return k3


def _make_k4(nhall, cph):
    @functools.partial(
        pl.kernel, mesh=_mesh, compiler_params=_cp,
        out_type=jax.ShapeDtypeStruct((2 * NP, HALF), f32),
        scratch_types=[
            pltpu.VMEM((24,), i32),
            pltpu.VMEM((K,), i32),
            pltpu.VMEM((K,), i32),
            pltpu.VMEM((K, 16), f32),
            pltpu.VMEM((K, 16), f32),
            pltpu.VMEM((K, HALF), f32),
            pltpu.VMEM((K, HALF), f32),
            pltpu.VMEM((HALF,), f32),
            pltpu.VMEM((640, HALF), f32),
            pltpu.SemaphoreType.DMA,
        ],
    )
    def k4(ex_ref, den_ref, xl_ref, src_ref, dsts_ref, bias_ref, offs_ref,
           outf_ref,
           offs_v, idxs_v, dst_v, ex_v, dch_v, xls_v, stgw_v, bias_v, ob_v,
           sem1):
        core = lax.axis_index("c")
        sub = lax.axis_index("s")
        iota = _iota()
        coff = jnp.full((16,), core * N, i32)
        zero16 = jnp.zeros((16,), f32)

        def zrow(j, _):
            rv = jnp.full((16,), j, i32)
            for t in range(8):
                plsc.store_scatter(ob_v, [rv, iota + 16 * t], zero16)
            return 0

        lax.fori_loop(0, 640, zrow, 0)

        pltpu.sync_copy(offs_ref, offs_v)
        estart = _sget(offs_v, sub)
        eend = _sget(offs_v, sub + 1)
        nbase = sub * 640
        e0al = (estart // K) * K
        nch = (eend - e0al + K - 1) // K

        def chunk(ci, _):
            e0 = e0al + ci * K
            pltpu.sync_copy(src_ref.at[pl.ds(e0, K)], idxs_v)
            pltpu.sync_copy(dsts_ref.at[pl.ds(e0, K)], dst_v)
            for g in range(K // 16):
                sl = pl.ds(g * 16, 16)
                idxs_v[sl] = idxs_v[sl] + coff
            cp1 = pltpu.async_copy(xl_ref.at[idxs_v], xls_v, sem1)
            pltpu.sync_copy(ex_ref.at[pl.ds(e0, K)], ex_v)
            cp1.wait()

            lo = jnp.maximum(estart - e0, 0)
            hi = jnp.minimum(eend - e0, K)

            def edge(r, _):
                rv = jnp.full((16,), r, i32)
                d = jnp.max(plsc.load_gather(dst_v, [rv]))
                lv = jnp.full((16,), d - nbase, i32)
                if nhall == 8:
                    exb = [plsc.load_gather(
                        ex_v, [rv, jnp.full((16,), lh + 4 * core, i32)])
                        for lh in range(4)]
                else:
                    exb = [plsc.load_gather(ex_v, [rv, jnp.full((16,), 0, i32)])]
                for t in range(8):
                    cols = iota + 16 * t
                    xv = plsc.load_gather(xls_v, [rv, cols])
                    cur = plsc.load_gather(ob_v, [lv, cols])
                    w = exb[(16 * t) // cph]
                    plsc.store_scatter(ob_v, [lv, cols], cur + xv * w)
                return 0

            lax.fori_loop(lo, hi, edge, 0)
            return 0

        lax.fori_loop(0, nch, chunk, 0)

        pltpu.sync_copy(bias_ref.at[pl.ds(core * HALF, HALF)], bias_v)
        bvecs = [bias_v[pl.ds(j * 16, 16)] for j in range(8)]
        nlh = 4 if nhall == 8 else 1

        def wb(cc, _):
            r0 = nbase + cc * K
            pltpu.sync_copy(den_ref.at[pl.ds(r0, K)], dch_v)

            def brow(i, _):
                rv = jnp.full((16,), i, i32)
                gl = jnp.full((16,), cc * K, i32) + rv
                rinv = []
                for lh in range(nlh):
                    h = lh + nlh * core if nhall == 8 else 0
                    dv = plsc.load_gather(dch_v, [rv, jnp.full((16,), h, i32)])
                    rinv.append(1.0 / (dv + 1e-16))
                for t in range(8):
                    cols = iota + 16 * t
                    cur = plsc.load_gather(ob_v, [gl, cols])
                    val = cur * rinv[t // (8 // nlh)] + bvecs[t]
                    plsc.store_scatter(stgw_v, [rv, cols], val)
                return 0

            lax.fori_loop(0, K, brow, 0)
            pltpu.sync_copy(stgw_v, outf_ref.at[pl.ds(core * NP + r0, K)])
            return 0

        lax.fori_loop(0, 10, wb, 0)

    return k4


_k2_split = _make_k2(4, 32)
_k2_sum = _make_k2(1, HALF)
_k3_split = _make_k3(True)
_k3_sum = _make_k3(False)
_k4_split = _make_k4(8, 32)
_k4_sum = _make_k4(1, HALF)


def kernel(x, edge_index, edge_attr, Wl1, bl1, We1, att1, bias1,
           Wl2, bl2, We2, att2, bias2,
           Wl3, bl3, Wr3, br3, We3, att3, bias3):
    src = edge_index[0].astype(i32)
    dst = edge_index[1].astype(i32)

    loop = jnp.arange(N, dtype=i32)
    padn = E2P - E2
    src2 = jnp.concatenate([src, loop, jnp.zeros((padn,), i32)])
    dst2 = jnp.concatenate([dst, loop, jnp.full((padn,), N, i32)])
    perm = jnp.argsort(dst2).astype(i32)
    dsts = dst2[perm]
    srcs = src2[perm]
    dstg = jnp.where(dsts == N, 0, dsts)
    isreal = (perm < E)
    eag = jnp.pad(edge_attr, ((0, 0), (0, 12)))[jnp.where(isreal, perm, 0)]
    eag = eag * isreal[:, None].astype(f32)
    eag = eag.at[:, 4].set(isreal.astype(f32))
    ext = jnp.zeros((EXT - E2P,), i32)
    srcs_x = jnp.concatenate([srcs, ext])
    dstg_x = jnp.concatenate([dstg, ext])
    dsts_x = jnp.concatenate([dsts, jnp.full((EXT - E2P,), N, i32)])
    perm_x = jnp.concatenate([perm, ext])
    eag_x = jnp.concatenate([eag, jnp.zeros((EXT - E2P, 16), f32)], axis=0)
    bounds32 = jnp.searchsorted(dsts, jnp.arange(0, NP + 1, 320, dtype=i32)
                                ).astype(i32)
    offs33 = jnp.concatenate([bounds32, jnp.zeros((7,), i32)])
    bounds16 = jnp.searchsorted(dsts, jnp.arange(0, NP + 1, 640, dtype=i32)
                                ).astype(i32)
    offs17 = jnp.concatenate([bounds16, jnp.zeros((7,), i32)])

    mean16 = _k0(eag_x, dsts_x, offs33)
    ea2p = jnp.concatenate(
        [jnp.pad(edge_attr, ((0, 0), (0, 12))), mean16[:N],
         jnp.zeros((padn, 16), f32)], axis=0)

    def att_bcast(att):
        return jnp.broadcast_to(att.reshape(HID)[:, None], (HID, 16))

    def gat_layer(xl, xr, We, att, bias, split):
        Wep = jnp.pad(We, ((0, 12), (0, 0)))
        es = _mm_edge(ea2p, Wep)
        attf = att_bcast(att)
        if split:
            alpha, mx = _k2_split(xl, xr, es, srcs_x, dstg_x, perm_x, attf)
            ex, den = _k3_split(alpha, dsts_x, mx, offs33)
            outf = _k4_split(ex, den, xl, srcs_x, dsts_x, bias, offs17)
        else:
            alpha, mx = _k2_sum(xl, xr, es, srcs_x, dstg_x, perm_x, attf)
            ex, den = _k3_sum(alpha, dsts_x, mx, offs33)
            outf = _k4_sum(ex, den, xl, srcs_x, dsts_x, bias, offs17)
        return outf

    xl1 = _mm_first(x, Wl1, bl1)
    h1 = gat_layer(xl1, xl1, We1, att1, bias1, True)
    xl2 = _mm_mid(h1, [Wl2], [bl2])
    h2 = gat_layer(xl2, xl2, We2, att2, bias2, True)
    xl3, xr3 = _mm_mid(h2, [Wl3, Wr3], [bl3, br3])
    h3 = gat_layer(xl3, xr3, We3, att3, bias3, False)

    return jnp.concatenate([h3[:N], h3[NP:NP + N]], axis=1)
